# baseline (speedup 1.0000x reference)
import functools

import jax
import jax.numpy as jnp
from jax import lax
from jax.experimental import pallas as pl
from jax.experimental.pallas import tpu as pltpu
from jax.experimental.pallas import tpu_sc as plsc

B, P, D = 4096, 128, 512
NC, NS, L = 2, 16, 16
NW = NC * NS
BPW = B // NW
CH = 4
BPC = BPW // CH
GPC = BPC // L
PV = P // L


def _make_kernel():
    mesh = plsc.VectorSubcoreMesh(core_axis_name="c", subcore_axis_name="s")

    @functools.partial(
        pl.kernel,
        mesh=mesh,
        compiler_params=pltpu.CompilerParams(needs_layout_passes=False),
        out_type=jax.ShapeDtypeStruct((B, 1, D), jnp.float32),
        scratch_types=[
            pltpu.VMEM((BPW, P), jnp.float32),
            pltpu.VMEM((BPW,), jnp.int32),
            pltpu.VMEM((BPW, 1, D), jnp.float32),
            [pltpu.SemaphoreType.DMA] * CH,
        ],
    )
    def sel(tgt_hbm, pi_hbm, out_hbm, pi_v, idx_v, rows_v, sems):
        wid = lax.axis_index("s") * NC + lax.axis_index("c")
        base = wid * BPW

        pltpu.sync_copy(pi_hbm.at[pl.ds(base, BPW)], pi_v)

        lane = lax.iota(jnp.int32, L)

        def make_group_body(c):
            def group_body(g, carry):
                res = jnp.zeros((L,), jnp.int32)
                for i in range(L):
                    b = c * BPC + g * L + i
                    m = pi_v[b, pl.ds(0, L)]
                    a = jnp.zeros((L,), jnp.int32)
                    for j in range(1, PV):
                        v = pi_v[b, pl.ds(j * L, L)]
                        a = jnp.where(v > m, j, a)
                        m = jnp.maximum(m, v)
                    mx = jnp.max(m)
                    cand = jnp.where(m == mx, a * L + lane, P)
                    p_sel = jnp.min(cand)
                    row = (base + b) * P + p_sel
                    res = jnp.where(lane == i, row, res)
                idx_v[pl.ds(c * BPC + g * L, L)] = res
                return carry

            return group_body

        gathers = []
        for c in range(CH):
            lax.fori_loop(0, GPC, make_group_body(c), 0)
            gathers.append(
                pltpu.async_copy(
                    tgt_hbm.at[idx_v.at[pl.ds(c * BPC, BPC)]],
                    rows_v.at[pl.ds(c * BPC, BPC), 0],
                    sems[c],
                )
            )

        writes = []
        for c in range(CH):
            gathers[c].wait()
            writes.append(
                pltpu.async_copy(
                    rows_v.at[pl.ds(c * BPC, BPC)],
                    out_hbm.at[pl.ds(base + c * BPC, BPC)],
                    sems[c],
                )
            )
        for w in writes:
            w.wait()

    return sel


_SEL = _make_kernel()


@jax.jit
def kernel(src, tgt, part_indicator):
    del src
    return _SEL(tgt.reshape(B * P, D), part_indicator)


# speedup vs baseline: 1.0320x; 1.0320x over previous
import functools

import jax
import jax.numpy as jnp
from jax import lax
from jax.experimental import pallas as pl
from jax.experimental.pallas import tpu as pltpu
from jax.experimental.pallas import tpu_sc as plsc

B, P, D = 4096, 128, 512
NC, NS, L = 2, 16, 16
NW = NC * NS
BPW = B // NW
CH = 4
BPC = BPW // CH
GROUPS = BPW // L
PV = P // L


def _make_kernel():
    mesh = plsc.VectorSubcoreMesh(core_axis_name="c", subcore_axis_name="s")

    @functools.partial(
        pl.kernel,
        mesh=mesh,
        compiler_params=pltpu.CompilerParams(needs_layout_passes=False),
        out_type=jax.ShapeDtypeStruct((B, 1, D), jnp.float32),
        scratch_types=[
            pltpu.VMEM((BPW, P), jnp.float32),
            pltpu.VMEM((BPW,), jnp.int32),
            pltpu.VMEM((BPW, 1, D), jnp.float32),
            [pltpu.SemaphoreType.DMA] * CH,
        ],
    )
    def sel(tgt_hbm, pi_hbm, out_hbm, pi_v, idx_v, rows_v, sems):
        wid = lax.axis_index("s") * NC + lax.axis_index("c")
        base = wid * BPW

        pltpu.sync_copy(pi_hbm.at[pl.ds(base, BPW)], pi_v)

        lane = lax.iota(jnp.int32, L)

        def group_body(g, carry):
            res = jnp.zeros((L,), jnp.int32)
            gbase = g * L
            for i in range(L):
                m = pi_v[gbase + i, pl.ds(0, L)]
                a = jnp.zeros((L,), jnp.int32)
                for j in range(1, PV):
                    v = pi_v[gbase + i, pl.ds(j * L, L)]
                    a = jnp.where(v > m, j, a)
                    m = jnp.maximum(m, v)
                mx = jnp.max(m)
                cand = jnp.where(m == mx, a * L + lane, P)
                p_sel = jnp.min(cand)
                row = (base + gbase + i) * P + p_sel
                res = jnp.where(lane == i, row, res)
            idx_v[pl.ds(gbase, L)] = res
            return carry

        def gather_chunk(c):
            return pltpu.async_copy(
                tgt_hbm.at[idx_v.at[pl.ds(c * BPC, BPC)]],
                rows_v.at[pl.ds(c * BPC, BPC), 0],
                sems[c],
            )

        lax.fori_loop(0, GROUPS // 2, group_body, 0)
        gathers = [gather_chunk(0), gather_chunk(1)]
        lax.fori_loop(GROUPS // 2, GROUPS, group_body, 0)
        gathers += [gather_chunk(2), gather_chunk(3)]

        writes = []
        for c in range(CH):
            gathers[c].wait()
            writes.append(
                pltpu.async_copy(
                    rows_v.at[pl.ds(c * BPC, BPC)],
                    out_hbm.at[pl.ds(base + c * BPC, BPC)],
                    sems[c],
                )
            )
        for w in writes:
            w.wait()

    return sel


_SEL = _make_kernel()


@jax.jit
def kernel(src, tgt, part_indicator):
    del src
    return _SEL(tgt.reshape(B * P, D), part_indicator)
